# baseline (speedup 1.0000x reference)
import functools

import jax
import jax.numpy as jnp
from jax import lax
from jax.experimental import pallas as pl
from jax.experimental.pallas import tpu as pltpu
from jax.experimental.pallas import tpu_sc as plsc

_N_HEADS = 16
_MAX_DIST = 512
_TBL = 2 * _MAX_DIST + 1
_Q = 2048
_K = 2048
_P_LEFT = _K - _MAX_DIST - 1
_E8_LEN = 4224
_E128_LEN = 4096
_U_VECS = _E8_LEN // 16


def _sc_prep_body(tbl_hbm, e8_hbm, tbl_v, e8s_v, sem):
    wid = lax.axis_index("s") * 2 + lax.axis_index("c")
    h = wid // 2
    half = wid % 2
    pltpu.sync_copy(tbl_hbm, tbl_v)
    dvec = tbl_v[pl.ds(_TBL * _N_HEADS, 16)].astype(jnp.int32)
    iota = lax.iota(jnp.int32, 16)
    unroll = 8
    for r_loc in range(4):
        r = 4 * half + r_loc
        base_vec = iota - r - _P_LEFT + dvec

        def body(t, carry, r_loc=r_loc, base_vec=base_vec):
            u0 = t * (16 * unroll)
            for k in range(unroll):
                u_base = u0 + 16 * k
                idx = jnp.clip(u_base + base_vec, 0, _TBL - 1) * _N_HEADS + h
                vals = plsc.load_gather(tbl_v, [idx])
                e8s_v[pl.ds(pl.multiple_of(r_loc * _E8_LEN + u_base, 8), 16)] = vals
            return carry

        lax.fori_loop(0, _U_VECS // unroll, body, 0)
    dst = pl.multiple_of((h * 8 + 4 * half) * _E8_LEN, 128)
    pltpu.sync_copy(e8s_v, e8_hbm.at[pl.ds(dst, 4 * _E8_LEN)])


def _expand_body(e8_ref, out_hbm, e128_ref, sem):
    h = pl.program_id(0)
    for a in range(16):
        off = 127 - 8 * a
        e128_ref[8 * a:8 * a + 8, :] = e8_ref[0, :, off:off + _E128_LEN]
    copies = []
    for b in range(16):
        s = 1920 - 128 * b
        copies.append(pltpu.make_async_copy(
            e128_ref.at[:, s:s + _K],
            out_hbm.at[h, pl.ds(128 * b, 128), :],
            sem))
    for cp in copies:
        cp.start()
    for cp in copies:
        cp.wait()


def kernel(q_len, k_len, rel_bias):
    d = (k_len - _K) - (q_len - _Q)

    sc_prep = functools.partial(
        pl.kernel,
        mesh=plsc.VectorSubcoreMesh(core_axis_name="c", subcore_axis_name="s"),
        out_type=jax.ShapeDtypeStruct((_N_HEADS * 8 * _E8_LEN,), rel_bias.dtype),
        scratch_types=[
            pltpu.VMEM((_TBL * _N_HEADS + 16,), rel_bias.dtype),
            pltpu.VMEM((4 * _E8_LEN,), rel_bias.dtype),
            pltpu.SemaphoreType.DMA,
        ],
        compiler_params=pltpu.CompilerParams(needs_layout_passes=False),
    )(_sc_prep_body)
    tbl_plus = jnp.concatenate(
        [rel_bias.reshape(-1), jnp.full((16,), d, rel_bias.dtype)])
    e8 = sc_prep(tbl_plus).reshape(_N_HEADS, 8, _E8_LEN)

    out = pl.pallas_call(
        _expand_body,
        grid=(_N_HEADS,),
        in_specs=[pl.BlockSpec((1, 8, _E8_LEN), lambda h: (h, 0, 0))],
        out_specs=pl.BlockSpec(memory_space=pl.ANY),
        out_shape=jax.ShapeDtypeStruct((_N_HEADS, _Q, _K), rel_bias.dtype),
        scratch_shapes=[pltpu.VMEM((128, _E128_LEN), rel_bias.dtype),
                        pltpu.SemaphoreType.DMA],
        compiler_params=pltpu.CompilerParams(
            dimension_semantics=("parallel",),
        ),
    )(e8)
    return out


# speedup vs baseline: 1.2391x; 1.2391x over previous
import functools

import jax
import jax.numpy as jnp
from jax import lax
from jax.experimental import pallas as pl
from jax.experimental.pallas import tpu as pltpu
from jax.experimental.pallas import tpu_sc as plsc

_N_HEADS = 16
_MAX_DIST = 512
_TBL = 2 * _MAX_DIST + 1
_Q = 2048
_K = 2048
_P_LEFT = _K - _MAX_DIST - 1
_E_SC = 4352
_E_HALF = _E_SC // 2
_E8_LEN = 4224
_E128_LEN = 4096


def _sc_prep_body(tbl_hbm, e_hbm, tbl_v, e_v, sem):
    wid = lax.axis_index("s") * 2 + lax.axis_index("c")
    h = wid // 2
    half = wid % 2
    pltpu.sync_copy(tbl_hbm, tbl_v)
    dvec = tbl_v[pl.ds(_TBL * _N_HEADS, 16)].astype(jnp.int32)
    iota = lax.iota(jnp.int32, 16)
    base_vec = half * _E_HALF + iota - (7 + _P_LEFT) + dvec
    unroll = 8

    def body(t, carry):
        u0 = t * (16 * unroll)
        for k in range(unroll):
            u_base = u0 + 16 * k
            idx = jnp.clip(u_base + base_vec, 0, _TBL - 1) * _N_HEADS + h
            vals = plsc.load_gather(tbl_v, [idx])
            e_v[pl.ds(pl.multiple_of(u_base, 8), 16)] = vals
        return carry

    lax.fori_loop(0, _E_HALF // (16 * unroll), body, 0)
    dst = pl.multiple_of(h * _E_SC + half * _E_HALF, 128)
    pltpu.sync_copy(e_v, e_hbm.at[pl.ds(dst, _E_HALF)])


def _expand_body(e_ref, out_ref, e8_ref, e128_ref):
    for r in range(8):
        e8_ref[r:r + 1, :] = e_ref[0, :, 7 - r:7 - r + _E8_LEN]
    for a in range(16):
        off = 127 - 8 * a
        e128_ref[8 * a:8 * a + 8, :] = e8_ref[:, off:off + _E128_LEN]
    for b in range(16):
        s = 1920 - 128 * b
        out_ref[0, 128 * b:128 * (b + 1), :] = e128_ref[:, s:s + _K]


def kernel(q_len, k_len, rel_bias):
    d = (k_len - _K) - (q_len - _Q)

    sc_prep = functools.partial(
        pl.kernel,
        mesh=plsc.VectorSubcoreMesh(core_axis_name="c", subcore_axis_name="s"),
        out_type=jax.ShapeDtypeStruct((_N_HEADS * _E_SC,), rel_bias.dtype),
        scratch_types=[
            pltpu.VMEM((_TBL * _N_HEADS + 16,), rel_bias.dtype),
            pltpu.VMEM((_E_HALF,), rel_bias.dtype),
            pltpu.SemaphoreType.DMA,
        ],
        compiler_params=pltpu.CompilerParams(needs_layout_passes=False),
    )(_sc_prep_body)
    tbl_plus = jnp.concatenate(
        [rel_bias.reshape(-1), jnp.full((16,), d, rel_bias.dtype)])
    e = sc_prep(tbl_plus).reshape(_N_HEADS, 1, _E_SC)

    out = pl.pallas_call(
        _expand_body,
        grid=(_N_HEADS,),
        in_specs=[pl.BlockSpec((1, 1, _E_SC), lambda h: (h, 0, 0))],
        out_specs=pl.BlockSpec((1, _Q, _K), lambda h: (h, 0, 0)),
        out_shape=jax.ShapeDtypeStruct((_N_HEADS, _Q, _K), rel_bias.dtype),
        scratch_shapes=[pltpu.VMEM((8, _E8_LEN), rel_bias.dtype),
                        pltpu.VMEM((128, _E128_LEN), rel_bias.dtype)],
        compiler_params=pltpu.CompilerParams(
            dimension_semantics=("parallel",),
        ),
    )(e)
    return out


# speedup vs baseline: 1.2879x; 1.0394x over previous
import functools

import jax
import jax.numpy as jnp
from jax import lax
from jax.experimental import pallas as pl
from jax.experimental.pallas import tpu as pltpu
from jax.experimental.pallas import tpu_sc as plsc

_N_HEADS = 16
_MAX_DIST = 512
_TBL = 2 * _MAX_DIST + 1
_Q = 2048
_K = 2048
_P_LEFT = _K - _MAX_DIST - 1
_E_SC = 4352
_E_HALF = _E_SC // 2
_E8_LEN = 4224
_E128_LEN = 4096


_ROW = 1056


def _sc_prep_body(tbl_hbm, e_hbm, tbl_v, e_v, sem):
    wid = lax.axis_index("s") * 2 + lax.axis_index("c")
    h = wid // 2
    half = wid % 2
    pltpu.sync_copy(tbl_hbm.at[pl.ds(pl.multiple_of(h * _ROW, 8), _ROW)], tbl_v)
    dvec = tbl_v[pl.ds(1032, 16)].astype(jnp.int32)
    iota = lax.iota(jnp.int32, 16)
    base_vec = half * _E_HALF + iota - (7 + _P_LEFT) + dvec
    unroll = 8

    def body(t, carry):
        u0 = t * (16 * unroll)
        for k in range(unroll):
            u_base = u0 + 16 * k
            idx = jnp.clip(u_base + base_vec, 0, _TBL - 1)
            vals = plsc.load_gather(tbl_v, [idx])
            e_v[pl.ds(pl.multiple_of(u_base, 8), 16)] = vals
        return carry

    lax.fori_loop(0, _E_HALF // (16 * unroll), body, 0)
    dst = pl.multiple_of(h * _E_SC + half * _E_HALF, 128)
    pltpu.sync_copy(e_v, e_hbm.at[pl.ds(dst, _E_HALF)])


def _expand_body(e_ref, out_ref, e8_ref, e128_ref):
    for r in range(8):
        e8_ref[r:r + 1, :] = e_ref[0, :, 7 - r:7 - r + _E8_LEN]
    for a in range(16):
        off = 127 - 8 * a
        e128_ref[8 * a:8 * a + 8, :] = e8_ref[:, off:off + _E128_LEN]
    for b in range(16):
        s = 1920 - 128 * b
        out_ref[0, 128 * b:128 * (b + 1), :] = e128_ref[:, s:s + _K]


def kernel(q_len, k_len, rel_bias):
    d = (k_len - _K) - (q_len - _Q)

    sc_prep = functools.partial(
        pl.kernel,
        mesh=plsc.VectorSubcoreMesh(core_axis_name="c", subcore_axis_name="s"),
        out_type=jax.ShapeDtypeStruct((_N_HEADS * _E_SC,), rel_bias.dtype),
        scratch_types=[
            pltpu.VMEM((_ROW,), rel_bias.dtype),
            pltpu.VMEM((_E_HALF,), rel_bias.dtype),
            pltpu.SemaphoreType.DMA,
        ],
        compiler_params=pltpu.CompilerParams(needs_layout_passes=False),
    )(_sc_prep_body)
    tbl_t = jnp.concatenate(
        [rel_bias.T, jnp.zeros((_N_HEADS, 7), rel_bias.dtype),
         jnp.full((_N_HEADS, 16), d, rel_bias.dtype),
         jnp.zeros((_N_HEADS, _ROW - 1048), rel_bias.dtype)], axis=1)
    e = sc_prep(tbl_t.reshape(-1)).reshape(_N_HEADS, 1, _E_SC)

    out = pl.pallas_call(
        _expand_body,
        grid=(_N_HEADS,),
        in_specs=[pl.BlockSpec((1, 1, _E_SC), lambda h: (h, 0, 0))],
        out_specs=pl.BlockSpec((1, _Q, _K), lambda h: (h, 0, 0)),
        out_shape=jax.ShapeDtypeStruct((_N_HEADS, _Q, _K), rel_bias.dtype),
        scratch_shapes=[pltpu.VMEM((8, _E8_LEN), rel_bias.dtype),
                        pltpu.VMEM((128, _E128_LEN), rel_bias.dtype)],
        compiler_params=pltpu.CompilerParams(
            dimension_semantics=("parallel",),
        ),
    )(e)
    return out


# speedup vs baseline: 1.2950x; 1.0055x over previous
import functools

import jax
import jax.numpy as jnp
from jax import lax
from jax.experimental import pallas as pl
from jax.experimental.pallas import tpu as pltpu
from jax.experimental.pallas import tpu_sc as plsc

_N_HEADS = 16
_MAX_DIST = 512
_TBL = 2 * _MAX_DIST + 1
_Q = 2048
_K = 2048
_P_LEFT = _K - _MAX_DIST - 1
_E_SC = 4352
_E_HALF = _E_SC // 2
_E8_LEN = 4224
_E128_LEN = 4096


_ROW = 1056


def _sc_prep_body(tbl_hbm, e_hbm, tbl_v, e_v, sem):
    wid = lax.axis_index("s") * 2 + lax.axis_index("c")
    h = wid // 2
    half = wid % 2
    pltpu.sync_copy(tbl_hbm.at[pl.ds(pl.multiple_of(h * _ROW, 8), _ROW)], tbl_v)
    dvec = tbl_v[pl.ds(1032, 16)].astype(jnp.int32)
    iota = lax.iota(jnp.int32, 16)
    base_vec = half * _E_HALF + iota - (7 + _P_LEFT) + dvec
    unroll = 8

    def body(t, carry):
        u0 = t * (16 * unroll)
        for k in range(unroll):
            u_base = u0 + 16 * k
            idx = jnp.clip(u_base + base_vec, 0, _TBL - 1)
            vals = plsc.load_gather(tbl_v, [idx])
            e_v[pl.ds(pl.multiple_of(u_base, 8), 16)] = vals
        return carry

    lax.fori_loop(0, _E_HALF // (16 * unroll), body, 0)
    dst = pl.multiple_of(h * _E_SC + half * _E_HALF, 128)
    pltpu.sync_copy(e_v, e_hbm.at[pl.ds(dst, _E_HALF)])


def _expand_body(e_ref, out_ref, e8_ref, e128_ref):
    for r in range(8):
        e8_ref[r:r + 1, :] = e_ref[0, :, 7 - r:7 - r + _E8_LEN]
    for a in range(16):
        off = 127 - 8 * a
        e128_ref[8 * a:8 * a + 8, :] = e8_ref[:, off:off + _E128_LEN]
    for qb in range(2):

        @pl.when(pl.program_id(1) == qb)
        def _():
            for b_loc in range(8):
                b = 8 * qb + b_loc
                s = 1920 - 128 * b
                out_ref[0, 128 * b_loc:128 * (b_loc + 1), :] = \
                    e128_ref[:, s:s + _K]


def kernel(q_len, k_len, rel_bias):
    d = (k_len - _K) - (q_len - _Q)

    sc_prep = functools.partial(
        pl.kernel,
        mesh=plsc.VectorSubcoreMesh(core_axis_name="c", subcore_axis_name="s"),
        out_type=jax.ShapeDtypeStruct((_N_HEADS * _E_SC,), rel_bias.dtype),
        scratch_types=[
            pltpu.VMEM((_ROW,), rel_bias.dtype),
            pltpu.VMEM((_E_HALF,), rel_bias.dtype),
            pltpu.SemaphoreType.DMA,
        ],
        compiler_params=pltpu.CompilerParams(needs_layout_passes=False),
    )(_sc_prep_body)
    tbl_t = jnp.concatenate(
        [rel_bias.T, jnp.zeros((_N_HEADS, 7), rel_bias.dtype),
         jnp.full((_N_HEADS, 16), d, rel_bias.dtype),
         jnp.zeros((_N_HEADS, _ROW - 1048), rel_bias.dtype)], axis=1)
    e = sc_prep(tbl_t.reshape(-1)).reshape(_N_HEADS, 1, _E_SC)

    out = pl.pallas_call(
        _expand_body,
        grid=(_N_HEADS, 2),
        in_specs=[pl.BlockSpec((1, 1, _E_SC), lambda h, qb: (h, 0, 0))],
        out_specs=pl.BlockSpec((1, _Q // 2, _K), lambda h, qb: (h, qb, 0)),
        out_shape=jax.ShapeDtypeStruct((_N_HEADS, _Q, _K), rel_bias.dtype),
        scratch_shapes=[pltpu.VMEM((8, _E8_LEN), rel_bias.dtype),
                        pltpu.VMEM((128, _E128_LEN), rel_bias.dtype)],
        compiler_params=pltpu.CompilerParams(
            dimension_semantics=("parallel", "parallel"),
        ),
    )(e)
    return out
